# baseline (speedup 1.0000x reference)
import jax
import jax.numpy as jnp
from jax.experimental import pallas as pl
from jax.experimental.pallas import tpu as pltpu


def _copy_body(idx_ref, in_ref, out_ref):
    out_ref[...] = in_ref[0]


def kernel(inputs, idx_x):
    b, n, h, w = inputs.shape
    gather_idx = (n - idx_x).astype(jnp.int32)

    grid_spec = pltpu.PrefetchScalarGridSpec(
        num_scalar_prefetch=1,
        grid=(b,),
        in_specs=[
            pl.BlockSpec((1, 1, h, w), lambda i, idx: (i, idx[i], 0, 0)),
        ],
        out_specs=pl.BlockSpec((1, h, w), lambda i, idx: (i, 0, 0)),
    )
    return pl.pallas_call(
        _copy_body,
        grid_spec=grid_spec,
        out_shape=jax.ShapeDtypeStruct((b, h, w), inputs.dtype),
    )(gather_idx, inputs)
